# Initial kernel scaffold; baseline (speedup 1.0000x reference)
import jax
import jax.numpy as jnp
from jax.experimental import pallas as pl


def kernel(x, coord, edge_index, edge_attr, W_e1, b_e1, W_e2, b_e2, W_c1, b_c1, W_c2, W_n1, b_n1, W_n2, b_n2, W_g, b_g):
    raise NotImplementedError("write your pallas kernel here")


# speedup vs baseline: 2.5673x; 2.5673x over previous
import functools

import jax
import jax.numpy as jnp
from jax import lax
from jax.experimental import pallas as pl
from jax.experimental.pallas import tpu as pltpu
from jax.experimental.pallas import tpu_sc as plsc

N = 10000
E = 320000
D = 128
DE = 16
W = 144
NC, NS = 2, 16
NW = NC * NS
PER_TILE = E // NW
C = 80
NCHUNK = PER_TILE // C
BE = 512
ROWS_PER_TILE = N // NS

_SC_PARAMS = pltpu.CompilerParams(use_tc_tiling_on_sc=False)


def _silu(v):
    return v * jax.nn.sigmoid(v)


def _tc_prep_body(x_ref, coord_ref, w1a_ref, w1b_ref, b1_ref, t1_ref, t2_ref):
    xb = x_ref[...]
    cpad = jnp.concatenate(
        [coord_ref[...], jnp.zeros((N, 13), jnp.float32)], axis=1)
    t1_ref[...] = jnp.concatenate(
        [jnp.dot(xb, w1a_ref[...], preferred_element_type=jnp.float32)
         + b1_ref[...], cpad], axis=1)
    t2_ref[...] = jnp.concatenate(
        [jnp.dot(xb, w1b_ref[...], preferred_element_type=jnp.float32),
         -cpad], axis=1)


def _tc_prep(x, coord, w1a, w1b, b1):
    return pl.pallas_call(
        _tc_prep_body,
        out_shape=[jax.ShapeDtypeStruct((N, W), jnp.float32),
                   jax.ShapeDtypeStruct((N, W), jnp.float32)],
    )(x, coord, w1a, w1b, b1)


def _sc_gather_body(t1_hbm, t2_hbm, src_hbm, dst_hbm, g_hbm,
                    idx1_v, idx2_v, buf1_v, buf2_v, sem1, sem2):
    wid = lax.axis_index("s") * NC + lax.axis_index("c")
    base = wid * PER_TILE

    def chunk(i, carry):
        off = base + i * C
        pltpu.sync_copy(src_hbm.at[pl.ds(off, C)], idx1_v)
        pltpu.sync_copy(dst_hbm.at[pl.ds(off, C)], idx2_v)
        cp1 = pltpu.async_copy(t1_hbm.at[idx1_v], buf1_v, sem1)
        cp2 = pltpu.async_copy(t2_hbm.at[idx2_v], buf2_v, sem2)
        cp1.wait()
        cp2.wait()

        def row(r, c2):
            for cc in range(W // 16):
                sl = pl.ds(cc * 16, 16)
                buf1_v[r, sl] = buf1_v[r, sl] + buf2_v[r, sl]
            return c2
        lax.fori_loop(0, C, row, 0)
        pltpu.sync_copy(buf1_v, g_hbm.at[pl.ds(off, C)])
        return carry

    lax.fori_loop(0, NCHUNK, chunk, 0)


def _sc_gather(t1, t2, src, dst):
    mesh = plsc.VectorSubcoreMesh(core_axis_name="c", subcore_axis_name="s")
    fn = pl.kernel(
        _sc_gather_body,
        out_type=jax.ShapeDtypeStruct((E, W), jnp.float32),
        mesh=mesh,
        compiler_params=_SC_PARAMS,
        scratch_types=[
            pltpu.VMEM((C,), jnp.int32),
            pltpu.VMEM((C,), jnp.int32),
            pltpu.VMEM((C, W), jnp.float32),
            pltpu.VMEM((C, W), jnp.float32),
            pltpu.SemaphoreType.DMA,
            pltpu.SemaphoreType.DMA,
        ],
    )
    return fn(t1, t2, src, dst)


def _tc_edge_body(g_ref, ea_ref, w1e_ref, wr_ref, we2_ref, be2_ref,
                  wc1_ref, bc1_ref, wc2_ref, r_ref):
    gb = g_ref[...]
    p = gb[:, :D]
    cd = gb[:, D:W]
    radial = jnp.sum(cd * cd, axis=1, keepdims=True)
    inv = 1.0 / (jnp.sqrt(radial) + 1e-30)
    xd = cd * inv
    z1 = p + radial * wr_ref[...] + jnp.dot(
        ea_ref[...], w1e_ref[...], preferred_element_type=jnp.float32)
    z1 = _silu(z1)
    z2 = _silu(jnp.dot(z1, we2_ref[...],
                       preferred_element_type=jnp.float32) + be2_ref[...])
    t = _silu(jnp.dot(z2, wc1_ref[...],
                      preferred_element_type=jnp.float32) + bc1_ref[...])
    w = jnp.dot(t, wc2_ref[...], preferred_element_type=jnp.float32)
    r_ref[...] = jnp.concatenate([z2, w * xd], axis=1)


def _tc_edge(g, edge_attr, w1e, wr, we2, be2, wc1, bc1, wc2):
    nblk = E // BE
    return pl.pallas_call(
        _tc_edge_body,
        grid=(nblk,),
        in_specs=[
            pl.BlockSpec((BE, W), lambda i: (i, 0)),
            pl.BlockSpec((BE, DE), lambda i: (i, 0)),
            pl.BlockSpec((DE, D), lambda i: (0, 0)),
            pl.BlockSpec((1, D), lambda i: (0, 0)),
            pl.BlockSpec((D, D), lambda i: (0, 0)),
            pl.BlockSpec((1, D), lambda i: (0, 0)),
            pl.BlockSpec((D, D), lambda i: (0, 0)),
            pl.BlockSpec((1, D), lambda i: (0, 0)),
            pl.BlockSpec((D, 1), lambda i: (0, 0)),
        ],
        out_specs=pl.BlockSpec((BE, W), lambda i: (i, 0)),
        out_shape=jax.ShapeDtypeStruct((E, W), jnp.float32),
    )(g, edge_attr, w1e, wr, we2, be2, wc1, bc1, wc2)


def _sc_scatter_body(r_hbm, dst_hbm, zeros_hbm, out_hbm,
                     idx_v, buf_v, accum_sh):
    c = lax.axis_index("c")
    s = lax.axis_index("s")
    base = (c * NS + s) * PER_TILE

    pltpu.sync_copy(zeros_hbm.at[pl.ds(s * ROWS_PER_TILE, ROWS_PER_TILE)],
                    accum_sh.at[pl.ds(s * ROWS_PER_TILE, ROWS_PER_TILE)])
    plsc.subcore_barrier()

    def chunk(i, carry):
        off = base + i * C
        pltpu.sync_copy(dst_hbm.at[pl.ds(off, C)], idx_v)
        pltpu.sync_copy(r_hbm.at[pl.ds(off, C)], buf_v)
        pltpu.sync_copy(buf_v, accum_sh.at[idx_v], add=True)
        return carry

    lax.fori_loop(0, NCHUNK, chunk, 0)
    plsc.subcore_barrier()
    pltpu.sync_copy(accum_sh.at[pl.ds(s * ROWS_PER_TILE, ROWS_PER_TILE)],
                    out_hbm.at[c, pl.ds(s * ROWS_PER_TILE, ROWS_PER_TILE)])


def _sc_scatter(r, dst, zeros):
    mesh = plsc.VectorSubcoreMesh(core_axis_name="c", subcore_axis_name="s")
    fn = pl.kernel(
        _sc_scatter_body,
        out_type=jax.ShapeDtypeStruct((NC, N, W), jnp.float32),
        mesh=mesh,
        compiler_params=_SC_PARAMS,
        scratch_types=[
            pltpu.VMEM((C,), jnp.int32),
            pltpu.VMEM((C, W), jnp.float32),
            pltpu.VMEM_SHARED((N, W), jnp.float32),
        ],
    )
    return fn(r, dst, zeros)


def _tc_node_body(x_ref, coord_ref, acc_ref, wn1a_ref, wn1b_ref, bn1_ref,
                  wn2_ref, bn2_ref, wg_ref, bg_ref,
                  hout_ref, xout_ref, pooled_ref):
    a0 = acc_ref[0]
    a1 = acc_ref[1]
    hn = a0[:, :D] + a1[:, :D]
    xn = a0[:, D:D + 3] + a1[:, D:D + 3]
    h1 = _silu(jnp.dot(x_ref[...], wn1a_ref[...],
                       preferred_element_type=jnp.float32)
               + jnp.dot(hn, wn1b_ref[...],
                         preferred_element_type=jnp.float32)
               + bn1_ref[...])
    h_out = jnp.dot(h1, wn2_ref[...],
                    preferred_element_type=jnp.float32) + bn2_ref[...]
    g = jnp.dot(h_out, wg_ref[...],
                preferred_element_type=jnp.float32) + bg_ref[...]
    m = jnp.max(g)
    ex = jnp.exp(g - m)
    gate = ex / jnp.sum(ex)
    pooled_ref[...] = jnp.sum(gate * h_out, axis=0, keepdims=True)
    hout_ref[...] = h_out
    xout_ref[...] = coord_ref[...] + xn


def _tc_node(x, coord, acc, wn1a, wn1b, bn1, wn2, bn2, wg, bg):
    return pl.pallas_call(
        _tc_node_body,
        out_shape=[jax.ShapeDtypeStruct((N, D), jnp.float32),
                   jax.ShapeDtypeStruct((N, 3), jnp.float32),
                   jax.ShapeDtypeStruct((1, D), jnp.float32)],
    )(x, coord, acc, wn1a, wn1b, bn1, wn2, bn2, wg, bg)


def kernel(x, coord, edge_index, edge_attr, W_e1, b_e1, W_e2, b_e2,
           W_c1, b_c1, W_c2, W_n1, b_n1, W_n2, b_n2, W_g, b_g):
    w1a = W_e1[:D]
    w1b = W_e1[D:2 * D]
    wr = W_e1[2 * D:2 * D + 1]
    w1e = W_e1[2 * D + 1:]
    src = edge_index[0]
    dst = edge_index[1]

    t1, t2 = _tc_prep(x, coord, w1a, w1b, b_e1.reshape(1, D))
    g = _sc_gather(t1, t2, src, dst)
    r = _tc_edge(g, edge_attr, w1e, wr, W_e2, b_e2.reshape(1, D),
                 W_c1, b_c1.reshape(1, D), W_c2)
    zeros = jnp.zeros((N, W), jnp.float32)
    acc = _sc_scatter(r, dst, zeros)
    h_out, x_out, pooled = _tc_node(
        x, coord, acc, W_n1[:D], W_n1[D:], b_n1.reshape(1, D),
        W_n2, b_n2.reshape(1, D), W_g, b_g.reshape(1, 1))
    return (h_out, x_out, pooled)


# speedup vs baseline: 3.0280x; 1.1795x over previous
import jax
import jax.numpy as jnp
from jax import lax
from jax.experimental import pallas as pl
from jax.experimental.pallas import tpu as pltpu
from jax.experimental.pallas import tpu_sc as plsc

N = 10000
E = 320000
D = 128
DE = 16
NC, NS = 2, 16
NW = NC * NS
PER_TILE = E // NW
C = 80
NCHUNK = PER_TILE // C
BE = 512
W2 = 16
ROWS_PER_TILE = N // NS

_SC_PARAMS = pltpu.CompilerParams(use_tc_tiling_on_sc=False,
                                  needs_layout_passes=False)


def _silu(v):
    return v * jax.nn.sigmoid(v)


def _tc_prep_body(x_ref, w1a_ref, w1b_ref, b1_ref, t1_ref, t2_ref):
    xb = x_ref[...]
    t1_ref[...] = jnp.dot(xb, w1a_ref[...],
                          preferred_element_type=jnp.float32) + b1_ref[...]
    t2_ref[...] = jnp.dot(xb, w1b_ref[...],
                          preferred_element_type=jnp.float32)


def _tc_prep(x, w1a, w1b, b1):
    return pl.pallas_call(
        _tc_prep_body,
        out_shape=[jax.ShapeDtypeStruct((N, D), jnp.float32),
                   jax.ShapeDtypeStruct((N, D), jnp.float32)],
    )(x, w1a, w1b, b1)


def _sc_gather_body(t1_hbm, t2_hbm, src_hbm, dst_hbm, coord_hbm,
                    g_hbm, d3_hbm,
                    idx1_v, idx2_v, buf1_v, buf2_v, coord_v, mp_v,
                    sem1, sem2):
    wid = lax.axis_index("s") * NC + lax.axis_index("c")
    base = wid * PER_TILE

    pltpu.sync_copy(coord_hbm, coord_v)

    def zmp(i, carry):
        mp_v[pl.ds(i * 16, 16)] = jnp.zeros((16,), jnp.float32)
        return carry
    lax.fori_loop(0, (4 * C) // 16, zmp, 0)

    def chunk(i, carry):
        off = base + i * C
        pltpu.sync_copy(src_hbm.at[pl.ds(off, C)], idx1_v)
        pltpu.sync_copy(dst_hbm.at[pl.ds(off, C)], idx2_v)
        cp1 = pltpu.async_copy(t1_hbm.at[idx1_v], buf1_v, sem1)
        cp2 = pltpu.async_copy(t2_hbm.at[idx2_v], buf2_v, sem2)
        cp1.wait()
        cp2.wait()

        def row(r, c2):
            for cc in range(D // 16):
                sl = pl.ds(cc * 16, 16)
                buf1_v[r, sl] = buf1_v[r, sl] + buf2_v[r, sl]
            return c2
        lax.fori_loop(0, C, row, 0)

        def grp(g, c3):
            gsl = pl.ds(g * 16, 16)
            s16 = idx1_v[gsl] * 4
            d16 = idx2_v[gsl] * 4
            l16 = (lax.iota(jnp.int32, 16) + g * 16) * 4
            for k in range(3):
                cs = plsc.load_gather(coord_v, [s16 + k])
                cd = plsc.load_gather(coord_v, [d16 + k])
                plsc.store_scatter(mp_v, [l16 + k], cs - cd)
            return c3
        lax.fori_loop(0, C // 16, grp, 0)

        pltpu.sync_copy(buf1_v, g_hbm.at[pl.ds(off, C)])
        pltpu.sync_copy(mp_v, d3_hbm.at[pl.ds(off * 4, C * 4)])
        return carry

    lax.fori_loop(0, NCHUNK, chunk, 0)


def _sc_gather(t1, t2, src, dst, coordp):
    mesh = plsc.VectorSubcoreMesh(core_axis_name="c", subcore_axis_name="s")
    fn = pl.kernel(
        _sc_gather_body,
        out_type=[jax.ShapeDtypeStruct((E, D), jnp.float32),
                  jax.ShapeDtypeStruct((4 * E,), jnp.float32)],
        mesh=mesh,
        compiler_params=_SC_PARAMS,
        scratch_types=[
            pltpu.VMEM((C,), jnp.int32),
            pltpu.VMEM((C,), jnp.int32),
            pltpu.VMEM((C, D), jnp.float32),
            pltpu.VMEM((C, D), jnp.float32),
            pltpu.VMEM((4 * N,), jnp.float32),
            pltpu.VMEM((4 * C,), jnp.float32),
            pltpu.SemaphoreType.DMA,
            pltpu.SemaphoreType.DMA,
        ],
    )
    return fn(t1, t2, src, dst, coordp)


def _tc_edge_body(g_ref, d3_ref, ea_ref, w1e_ref, wr_ref, we2_ref, be2_ref,
                  wc1_ref, bc1_ref, wc2_ref, r_ref, r2_ref):
    cd = d3_ref[...]
    radial = jnp.sum(cd * cd, axis=1, keepdims=True)
    inv = 1.0 / (jnp.sqrt(radial) + 1e-30)
    z1 = g_ref[...] + radial * wr_ref[...] + jnp.dot(
        ea_ref[...], w1e_ref[...], preferred_element_type=jnp.float32)
    z1 = _silu(z1)
    z2 = _silu(jnp.dot(z1, we2_ref[...],
                       preferred_element_type=jnp.float32) + be2_ref[...])
    t = _silu(jnp.dot(z2, wc1_ref[...],
                      preferred_element_type=jnp.float32) + bc1_ref[...])
    w = jnp.dot(t, wc2_ref[...], preferred_element_type=jnp.float32)
    r_ref[...] = z2
    r2_ref[...] = jnp.concatenate(
        [(w * inv) * cd, jnp.zeros((BE, 12), jnp.float32)], axis=1)


def _tc_edge(g, d3, edge_attr, w1e, wr, we2, be2, wc1, bc1, wc2):
    nblk = E // BE
    return pl.pallas_call(
        _tc_edge_body,
        grid=(nblk,),
        in_specs=[
            pl.BlockSpec((BE, D), lambda i: (i, 0)),
            pl.BlockSpec((BE, 4), lambda i: (i, 0)),
            pl.BlockSpec((BE, DE), lambda i: (i, 0)),
            pl.BlockSpec((DE, D), lambda i: (0, 0)),
            pl.BlockSpec((1, D), lambda i: (0, 0)),
            pl.BlockSpec((D, D), lambda i: (0, 0)),
            pl.BlockSpec((1, D), lambda i: (0, 0)),
            pl.BlockSpec((D, D), lambda i: (0, 0)),
            pl.BlockSpec((1, D), lambda i: (0, 0)),
            pl.BlockSpec((D, 1), lambda i: (0, 0)),
        ],
        out_specs=[pl.BlockSpec((BE, D), lambda i: (i, 0)),
                   pl.BlockSpec((BE, W2), lambda i: (i, 0))],
        out_shape=[jax.ShapeDtypeStruct((E, D), jnp.float32),
                   jax.ShapeDtypeStruct((E, W2), jnp.float32)],
    )(g, d3, edge_attr, w1e, wr, we2, be2, wc1, bc1, wc2)


def _sc_scatter_body(r_hbm, r2_hbm, dst_hbm, zh_hbm, zx_hbm,
                     outh_hbm, outx_hbm,
                     idx_v, bufh_v, bufx_v, acch_sh, accx_sh):
    c = lax.axis_index("c")
    s = lax.axis_index("s")
    base = (c * NS + s) * PER_TILE
    rsl = pl.ds(s * ROWS_PER_TILE, ROWS_PER_TILE)

    pltpu.sync_copy(zh_hbm.at[rsl], acch_sh.at[rsl])
    pltpu.sync_copy(zx_hbm.at[rsl], accx_sh.at[rsl])
    plsc.subcore_barrier()

    def chunk(i, carry):
        off = base + i * C
        pltpu.sync_copy(dst_hbm.at[pl.ds(off, C)], idx_v)
        pltpu.sync_copy(r_hbm.at[pl.ds(off, C)], bufh_v)
        pltpu.sync_copy(r2_hbm.at[pl.ds(off, C)], bufx_v)
        pltpu.sync_copy(bufh_v, acch_sh.at[idx_v], add=True)
        pltpu.sync_copy(bufx_v, accx_sh.at[idx_v], add=True)
        return carry

    lax.fori_loop(0, NCHUNK, chunk, 0)
    plsc.subcore_barrier()
    pltpu.sync_copy(acch_sh.at[rsl], outh_hbm.at[c, rsl])
    pltpu.sync_copy(accx_sh.at[rsl], outx_hbm.at[c, rsl])


def _sc_scatter(r, r2, dst, zh, zx):
    mesh = plsc.VectorSubcoreMesh(core_axis_name="c", subcore_axis_name="s")
    fn = pl.kernel(
        _sc_scatter_body,
        out_type=[jax.ShapeDtypeStruct((NC, N, D), jnp.float32),
                  jax.ShapeDtypeStruct((NC, N, W2), jnp.float32)],
        mesh=mesh,
        compiler_params=_SC_PARAMS,
        scratch_types=[
            pltpu.VMEM((C,), jnp.int32),
            pltpu.VMEM((C, D), jnp.float32),
            pltpu.VMEM((C, W2), jnp.float32),
            pltpu.VMEM_SHARED((N, D), jnp.float32),
            pltpu.VMEM_SHARED((N, W2), jnp.float32),
        ],
    )
    return fn(r, r2, dst, zh, zx)


def _tc_node_body(x_ref, coord_ref, acch_ref, accx_ref, wn1a_ref, wn1b_ref,
                  bn1_ref, wn2_ref, bn2_ref, wg_ref, bg_ref,
                  hout_ref, xout_ref, pooled_ref):
    hn = acch_ref[0] + acch_ref[1]
    xn = accx_ref[0][:, :3] + accx_ref[1][:, :3]
    h1 = _silu(jnp.dot(x_ref[...], wn1a_ref[...],
                       preferred_element_type=jnp.float32)
               + jnp.dot(hn, wn1b_ref[...],
                         preferred_element_type=jnp.float32)
               + bn1_ref[...])
    h_out = jnp.dot(h1, wn2_ref[...],
                    preferred_element_type=jnp.float32) + bn2_ref[...]
    g = jnp.dot(h_out, wg_ref[...],
                preferred_element_type=jnp.float32) + bg_ref[...]
    m = jnp.max(g)
    ex = jnp.exp(g - m)
    gate = ex / jnp.sum(ex)
    pooled_ref[...] = jnp.sum(gate * h_out, axis=0, keepdims=True)
    hout_ref[...] = h_out
    xout_ref[...] = coord_ref[...] + xn


def _tc_node(x, coord, acch, accx, wn1a, wn1b, bn1, wn2, bn2, wg, bg):
    return pl.pallas_call(
        _tc_node_body,
        out_shape=[jax.ShapeDtypeStruct((N, D), jnp.float32),
                   jax.ShapeDtypeStruct((N, 3), jnp.float32),
                   jax.ShapeDtypeStruct((1, D), jnp.float32)],
    )(x, coord, acch, accx, wn1a, wn1b, bn1, wn2, bn2, wg, bg)


def kernel(x, coord, edge_index, edge_attr, W_e1, b_e1, W_e2, b_e2,
           W_c1, b_c1, W_c2, W_n1, b_n1, W_n2, b_n2, W_g, b_g):
    w1a = W_e1[:D]
    w1b = W_e1[D:2 * D]
    wr = W_e1[2 * D:2 * D + 1]
    w1e = W_e1[2 * D + 1:]
    src = edge_index[0]
    dst = edge_index[1]
    coordp = jnp.pad(coord, ((0, 0), (0, 1))).reshape(-1)

    t1, t2 = _tc_prep(x, w1a, w1b, b_e1.reshape(1, D))
    g, d3 = _sc_gather(t1, t2, src, dst, coordp)
    r, r2 = _tc_edge(g, d3.reshape(E, 4), edge_attr, w1e, wr, W_e2,
                     b_e2.reshape(1, D), W_c1, b_c1.reshape(1, D), W_c2)
    zh = jnp.zeros((N, D), jnp.float32)
    zx = jnp.zeros((N, W2), jnp.float32)
    acch, accx = _sc_scatter(r, r2, dst, zh, zx)
    h_out, x_out, pooled = _tc_node(
        x, coord, acch, accx, W_n1[:D], W_n1[D:], b_n1.reshape(1, D),
        W_n2, b_n2.reshape(1, D), W_g, b_g.reshape(1, 1))
    return (h_out, x_out, pooled)


# speedup vs baseline: 3.5515x; 1.1729x over previous
import jax
import jax.numpy as jnp
from jax import lax
from jax.experimental import pallas as pl
from jax.experimental.pallas import tpu as pltpu
from jax.experimental.pallas import tpu_sc as plsc

N = 10000
E = 320000
D = 128
DE = 16
NC, NS = 2, 16
NW = NC * NS
PER_TILE = E // NW
C = 80
NCHUNK = PER_TILE // C
BE = 512
W2 = 16
ROWS_PER_TILE = N // NS

_SC_PARAMS = pltpu.CompilerParams(use_tc_tiling_on_sc=False,
                                  needs_layout_passes=False)


def _silu(v):
    return v * jax.nn.sigmoid(v)


def _tc_prep_body(x_ref, w1a_ref, w1b_ref, b1_ref, t1_ref, t2_ref):
    xb = x_ref[...]
    t1_ref[...] = jnp.dot(xb, w1a_ref[...],
                          preferred_element_type=jnp.float32) + b1_ref[...]
    t2_ref[...] = jnp.dot(xb, w1b_ref[...],
                          preferred_element_type=jnp.float32)


def _tc_prep(x, w1a, w1b, b1):
    return pl.pallas_call(
        _tc_prep_body,
        out_shape=[jax.ShapeDtypeStruct((N, D), jnp.float32),
                   jax.ShapeDtypeStruct((N, D), jnp.float32)],
    )(x, w1a, w1b, b1)


def _sc_gather_body(t1_hbm, t2_hbm, src_hbm, dst_hbm, coord_hbm,
                    g_hbm, d3_hbm, rad_hbm,
                    idx1_v, idx2_v, buf1_v, buf2_v, coord_v, mp_v, rad_v,
                    sem1, sem2):
    wid = lax.axis_index("s") * NC + lax.axis_index("c")
    base = wid * PER_TILE

    pltpu.sync_copy(coord_hbm, coord_v)

    def chunk(i, carry):
        off = base + i * C
        pltpu.sync_copy(src_hbm.at[pl.ds(off, C)], idx1_v)
        pltpu.sync_copy(dst_hbm.at[pl.ds(off, C)], idx2_v)
        cp1 = pltpu.async_copy(t1_hbm.at[idx1_v], buf1_v, sem1)
        cp2 = pltpu.async_copy(t2_hbm.at[idx2_v], buf2_v, sem2)
        cp1.wait()
        cp2.wait()

        def row(r, c2):
            for cc in range(D // 16):
                sl = pl.ds(cc * 16, 16)
                buf1_v[r, sl] = buf1_v[r, sl] + buf2_v[r, sl]
            return c2
        lax.fori_loop(0, C, row, 0)

        def grp(g, c3):
            gsl = pl.ds(g * 16, 16)
            s16 = idx1_v[gsl] * 4
            d16 = idx2_v[gsl] * 4
            l16 = (lax.iota(jnp.int32, 16) + g * 16) * 4
            dif = []
            for k in range(3):
                cs = plsc.load_gather(coord_v, [s16 + k])
                cd = plsc.load_gather(coord_v, [d16 + k])
                dif.append(cs - cd)
                plsc.store_scatter(mp_v, [l16 + k], dif[k])
            rad_v[gsl] = dif[0] * dif[0] + dif[1] * dif[1] + dif[2] * dif[2]
            return c3
        lax.fori_loop(0, C // 16, grp, 0)

        pltpu.sync_copy(buf1_v, g_hbm.at[pl.ds(off, C)])
        pltpu.sync_copy(mp_v, d3_hbm.at[pl.ds(off * 4, C * 4)])
        pltpu.sync_copy(rad_v, rad_hbm.at[pl.ds(off, C)])
        return carry

    lax.fori_loop(0, NCHUNK, chunk, 0)


def _sc_gather(t1, t2, src, dst, coordp):
    mesh = plsc.VectorSubcoreMesh(core_axis_name="c", subcore_axis_name="s")
    fn = pl.kernel(
        _sc_gather_body,
        out_type=[jax.ShapeDtypeStruct((E, D), jnp.float32),
                  jax.ShapeDtypeStruct((4 * E,), jnp.float32),
                  jax.ShapeDtypeStruct((E,), jnp.float32)],
        mesh=mesh,
        compiler_params=_SC_PARAMS,
        scratch_types=[
            pltpu.VMEM((C,), jnp.int32),
            pltpu.VMEM((C,), jnp.int32),
            pltpu.VMEM((C, D), jnp.float32),
            pltpu.VMEM((C, D), jnp.float32),
            pltpu.VMEM((4 * N,), jnp.float32),
            pltpu.VMEM((4 * C,), jnp.float32),
            pltpu.VMEM((C,), jnp.float32),
            pltpu.SemaphoreType.DMA,
            pltpu.SemaphoreType.DMA,
        ],
    )
    return fn(t1, t2, src, dst, coordp)


def _tc_edge_body(g_ref, radp_ref, ea_ref, w1e_ref, wr_ref, we2_ref, be2_ref,
                  wc1_ref, bc1_ref, wc2_ref, r_ref, wq_ref):
    radial = radp_ref[...].reshape(BE, 1)
    inv = 1.0 / (jnp.sqrt(radial) + 1e-30)
    eac = lax.dot_general(ea_ref[...], w1e_ref[...],
                          (((0,), (0,)), ((), ())),
                          preferred_element_type=jnp.float32)
    z1 = g_ref[...] + radial * wr_ref[...] + eac
    z1 = _silu(z1)
    z2 = _silu(jnp.dot(z1, we2_ref[...],
                       preferred_element_type=jnp.float32) + be2_ref[...])
    t = _silu(jnp.dot(z2, wc1_ref[...],
                      preferred_element_type=jnp.float32) + bc1_ref[...])
    w = jnp.dot(t, wc2_ref[...], preferred_element_type=jnp.float32)
    r_ref[...] = z2
    wq_ref[...] = (w * inv).reshape(BE)


def _tc_edge(g, radp, edge_attr, w1e, wr, we2, be2, wc1, bc1, wc2):
    nblk = E // BE
    return pl.pallas_call(
        _tc_edge_body,
        grid=(nblk,),
        in_specs=[
            pl.BlockSpec((BE, D), lambda i: (i, 0)),
            pl.BlockSpec((BE,), lambda i: (i,)),
            pl.BlockSpec((DE, BE), lambda i: (0, i)),
            pl.BlockSpec((DE, D), lambda i: (0, 0)),
            pl.BlockSpec((1, D), lambda i: (0, 0)),
            pl.BlockSpec((D, D), lambda i: (0, 0)),
            pl.BlockSpec((1, D), lambda i: (0, 0)),
            pl.BlockSpec((D, D), lambda i: (0, 0)),
            pl.BlockSpec((1, D), lambda i: (0, 0)),
            pl.BlockSpec((D, 1), lambda i: (0, 0)),
        ],
        out_specs=[pl.BlockSpec((BE, D), lambda i: (i, 0)),
                   pl.BlockSpec((BE,), lambda i: (i,))],
        out_shape=[jax.ShapeDtypeStruct((E, D), jnp.float32),
                   jax.ShapeDtypeStruct((E,), jnp.float32)],
    )(g, radp, edge_attr, w1e, wr, we2, be2, wc1, bc1, wc2)


def _sc_scatter_body(r_hbm, wq_hbm, d3_hbm, dst_hbm, zh_hbm, zx_hbm,
                     outh_hbm, outx_hbm,
                     idx_v, bufh_v, bufx_v, wq_v, d3_v, acch_sh, accx_sh):
    c = lax.axis_index("c")
    s = lax.axis_index("s")
    base = (c * NS + s) * PER_TILE
    rsl = pl.ds(s * ROWS_PER_TILE, ROWS_PER_TILE)

    pltpu.sync_copy(zh_hbm.at[rsl], acch_sh.at[rsl])
    pltpu.sync_copy(zx_hbm.at[rsl], accx_sh.at[rsl])
    plsc.subcore_barrier()

    def zx(r, carry):
        bufx_v[r, pl.ds(0, 16)] = jnp.zeros((16,), jnp.float32)
        return carry
    lax.fori_loop(0, C, zx, 0)

    def chunk(i, carry):
        off = base + i * C
        pltpu.sync_copy(dst_hbm.at[pl.ds(off, C)], idx_v)
        pltpu.sync_copy(r_hbm.at[pl.ds(off, C)], bufh_v)
        pltpu.sync_copy(wq_hbm.at[pl.ds(off, C)], wq_v)
        pltpu.sync_copy(d3_hbm.at[pl.ds(off * 4, C * 4)], d3_v)

        def grp(g, c2):
            gsl = pl.ds(g * 16, 16)
            w16 = wq_v[gsl]
            l16 = lax.iota(jnp.int32, 16) + g * 16
            l4 = l16 * 4
            for k in range(3):
                dk = plsc.load_gather(d3_v, [l4 + k])
                plsc.store_scatter(bufx_v, [l16, jnp.full((16,), k, jnp.int32)],
                                   w16 * dk)
            return c2
        lax.fori_loop(0, C // 16, grp, 0)

        pltpu.sync_copy(bufh_v, acch_sh.at[idx_v], add=True)
        pltpu.sync_copy(bufx_v, accx_sh.at[idx_v], add=True)
        return carry

    lax.fori_loop(0, NCHUNK, chunk, 0)
    plsc.subcore_barrier()
    pltpu.sync_copy(acch_sh.at[rsl], outh_hbm.at[c, rsl])
    pltpu.sync_copy(accx_sh.at[rsl], outx_hbm.at[c, rsl])


def _sc_scatter(r, wq, d3, dst, zh, zx):
    mesh = plsc.VectorSubcoreMesh(core_axis_name="c", subcore_axis_name="s")
    fn = pl.kernel(
        _sc_scatter_body,
        out_type=[jax.ShapeDtypeStruct((NC, N, D), jnp.float32),
                  jax.ShapeDtypeStruct((NC, N, W2), jnp.float32)],
        mesh=mesh,
        compiler_params=_SC_PARAMS,
        scratch_types=[
            pltpu.VMEM((C,), jnp.int32),
            pltpu.VMEM((C, D), jnp.float32),
            pltpu.VMEM((C, W2), jnp.float32),
            pltpu.VMEM((C,), jnp.float32),
            pltpu.VMEM((4 * C,), jnp.float32),
            pltpu.VMEM_SHARED((N, D), jnp.float32),
            pltpu.VMEM_SHARED((N, W2), jnp.float32),
        ],
    )
    return fn(r, wq, d3, dst, zh, zx)


def _tc_node_body(x_ref, coord_ref, acch_ref, accx_ref, wn1a_ref, wn1b_ref,
                  bn1_ref, wn2_ref, bn2_ref, wg_ref, bg_ref,
                  hout_ref, xout_ref, pooled_ref):
    hn = acch_ref[0] + acch_ref[1]
    xn = accx_ref[0][:, :3] + accx_ref[1][:, :3]
    h1 = _silu(jnp.dot(x_ref[...], wn1a_ref[...],
                       preferred_element_type=jnp.float32)
               + jnp.dot(hn, wn1b_ref[...],
                         preferred_element_type=jnp.float32)
               + bn1_ref[...])
    h_out = jnp.dot(h1, wn2_ref[...],
                    preferred_element_type=jnp.float32) + bn2_ref[...]
    g = jnp.dot(h_out, wg_ref[...],
                preferred_element_type=jnp.float32) + bg_ref[...]
    m = jnp.max(g)
    ex = jnp.exp(g - m)
    gate = ex / jnp.sum(ex)
    pooled_ref[...] = jnp.sum(gate * h_out, axis=0, keepdims=True)
    hout_ref[...] = h_out
    xout_ref[...] = coord_ref[...] + xn


def _tc_node(x, coord, acch, accx, wn1a, wn1b, bn1, wn2, bn2, wg, bg):
    return pl.pallas_call(
        _tc_node_body,
        out_shape=[jax.ShapeDtypeStruct((N, D), jnp.float32),
                   jax.ShapeDtypeStruct((N, 3), jnp.float32),
                   jax.ShapeDtypeStruct((1, D), jnp.float32)],
    )(x, coord, acch, accx, wn1a, wn1b, bn1, wn2, bn2, wg, bg)


def kernel(x, coord, edge_index, edge_attr, W_e1, b_e1, W_e2, b_e2,
           W_c1, b_c1, W_c2, W_n1, b_n1, W_n2, b_n2, W_g, b_g):
    w1a = W_e1[:D]
    w1b = W_e1[D:2 * D]
    wr = W_e1[2 * D:2 * D + 1]
    w1e = W_e1[2 * D + 1:]
    src = edge_index[0]
    dst = edge_index[1]
    coordp = jnp.pad(coord, ((0, 0), (0, 1))).reshape(-1)

    t1, t2 = _tc_prep(x, w1a, w1b, b_e1.reshape(1, D))
    g, d3, rad = _sc_gather(t1, t2, src, dst, coordp)
    r, wq = _tc_edge(g, rad, edge_attr.T, w1e, wr, W_e2,
                     b_e2.reshape(1, D), W_c1, b_c1.reshape(1, D), W_c2)
    zh = jnp.zeros((N, D), jnp.float32)
    zx = jnp.zeros((N, W2), jnp.float32)
    acch, accx = _sc_scatter(r, wq, d3, dst, zh, zx)
    h_out, x_out, pooled = _tc_node(
        x, coord, acch, accx, W_n1[:D], W_n1[D:], b_n1.reshape(1, D),
        W_n2, b_n2.reshape(1, D), W_g, b_g.reshape(1, 1))
    return (h_out, x_out, pooled)


# speedup vs baseline: 4.9378x; 1.3903x over previous
import jax
import jax.numpy as jnp
from jax import lax
from jax.experimental import pallas as pl
from jax.experimental.pallas import tpu as pltpu
from jax.experimental.pallas import tpu_sc as plsc

N = 10000
E = 320000
D = 128
DE = 16
NC, NS = 2, 16
NW = NC * NS
PER_TILE = E // NW
C = 80
NCHUNK = PER_TILE // C
BE = 512
W2 = 16
ROWS_PER_TILE = N // NS

_SC_PARAMS = pltpu.CompilerParams(use_tc_tiling_on_sc=False,
                                  needs_layout_passes=False)


def _silu(v):
    return v * jax.nn.sigmoid(v)


def _tc_prep_body(x_ref, w1a_ref, w1b_ref, b1_ref, t1_ref, t2_ref):
    xb = x_ref[...]
    t1_ref[...] = jnp.dot(xb, w1a_ref[...],
                          preferred_element_type=jnp.float32) + b1_ref[...]
    t2_ref[...] = jnp.dot(xb, w1b_ref[...],
                          preferred_element_type=jnp.float32)


def _tc_prep(x, w1a, w1b, b1):
    return pl.pallas_call(
        _tc_prep_body,
        out_shape=[jax.ShapeDtypeStruct((N, D), jnp.float32),
                   jax.ShapeDtypeStruct((N, D), jnp.float32)],
    )(x, w1a, w1b, b1)


def _sc_gather_body(t1_hbm, t2_hbm, src_hbm, dst_hbm, coord_hbm,
                    g_hbm, d3_hbm, rad_hbm,
                    idx1_v, idx2_v, buf1_v, buf2_v, coord_v, mp_v, rad_v,
                    sem1, sem2):
    wid = lax.axis_index("s") * NC + lax.axis_index("c")
    base = wid * PER_TILE

    pltpu.sync_copy(coord_hbm, coord_v)

    def issue(i, par):
        off = base + i * C
        pltpu.sync_copy(src_hbm.at[pl.ds(off, C)], idx1_v.at[par])
        pltpu.sync_copy(dst_hbm.at[pl.ds(off, C)], idx2_v.at[par])
        pltpu.async_copy(t1_hbm.at[idx1_v.at[par]], buf1_v.at[par],
                         sem1.at[par])
        pltpu.async_copy(t2_hbm.at[idx2_v.at[par]], buf2_v.at[par],
                         sem2.at[par])

    def process(i, par):
        off = base + i * C
        pltpu.make_async_copy(t1_hbm.at[idx1_v.at[par]],
                              buf1_v.at[par], sem1.at[par]).wait()
        pltpu.make_async_copy(t2_hbm.at[idx2_v.at[par]],
                              buf2_v.at[par], sem2.at[par]).wait()

        def row(r, c2):
            for cc in range(D // 16):
                sl = pl.ds(cc * 16, 16)
                buf1_v[par, r, sl] = buf1_v[par, r, sl] + buf2_v[par, r, sl]
            return c2
        lax.fori_loop(0, C, row, 0)

        def grp(g, c3):
            gsl = pl.ds(g * 16, 16)
            s16 = idx1_v[par, gsl] * 4
            d16 = idx2_v[par, gsl] * 4
            l16 = (lax.iota(jnp.int32, 16) + g * 16) * 4
            dif = []
            for k in range(3):
                cs = plsc.load_gather(coord_v, [s16 + k])
                cd = plsc.load_gather(coord_v, [d16 + k])
                dif.append(cs - cd)
                plsc.store_scatter(mp_v, [l16 + k], dif[k])
            rad_v[gsl] = dif[0] * dif[0] + dif[1] * dif[1] + dif[2] * dif[2]
            return c3
        lax.fori_loop(0, C // 16, grp, 0)

        pltpu.sync_copy(buf1_v.at[par], g_hbm.at[pl.ds(off, C)])
        pltpu.sync_copy(mp_v, d3_hbm.at[pl.ds(off * 4, C * 4)])
        pltpu.sync_copy(rad_v, rad_hbm.at[pl.ds(off, C)])

    issue(0, 0)

    def pair(j, carry):
        i0 = j * 2
        issue(i0 + 1, 1)
        process(i0, 0)
        issue(i0 + 2, 0)
        process(i0 + 1, 1)
        return carry

    lax.fori_loop(0, NCHUNK // 2 - 1, pair, 0)
    i0 = NCHUNK - 3
    issue(i0 + 1, 1)
    process(i0, 0)
    issue(i0 + 2, 0)
    process(i0 + 1, 1)
    process(i0 + 2, 0)


def _sc_gather(t1, t2, src, dst, coordp):
    mesh = plsc.VectorSubcoreMesh(core_axis_name="c", subcore_axis_name="s")
    fn = pl.kernel(
        _sc_gather_body,
        out_type=[jax.ShapeDtypeStruct((E, D), jnp.float32),
                  jax.ShapeDtypeStruct((4 * E,), jnp.float32),
                  jax.ShapeDtypeStruct((E,), jnp.float32)],
        mesh=mesh,
        compiler_params=_SC_PARAMS,
        scratch_types=[
            pltpu.VMEM((2, C), jnp.int32),
            pltpu.VMEM((2, C), jnp.int32),
            pltpu.VMEM((2, C, D), jnp.float32),
            pltpu.VMEM((2, C, D), jnp.float32),
            pltpu.VMEM((4 * N,), jnp.float32),
            pltpu.VMEM((4 * C,), jnp.float32),
            pltpu.VMEM((C,), jnp.float32),
            pltpu.SemaphoreType.DMA((2,)),
            pltpu.SemaphoreType.DMA((2,)),
        ],
    )
    return fn(t1, t2, src, dst, coordp)


def _tc_edge_body(g_ref, radp_ref, ea_ref, w1e_ref, wr_ref, we2_ref, be2_ref,
                  wc1_ref, bc1_ref, wc2_ref, r_ref, wq_ref):
    radial = radp_ref[...].reshape(BE, 1)
    inv = 1.0 / (jnp.sqrt(radial) + 1e-30)
    eac = lax.dot_general(ea_ref[...], w1e_ref[...],
                          (((0,), (0,)), ((), ())),
                          preferred_element_type=jnp.float32)
    z1 = g_ref[...] + radial * wr_ref[...] + eac
    z1 = _silu(z1)
    z2 = _silu(jnp.dot(z1, we2_ref[...],
                       preferred_element_type=jnp.float32) + be2_ref[...])
    t = _silu(jnp.dot(z2, wc1_ref[...],
                      preferred_element_type=jnp.float32) + bc1_ref[...])
    w = jnp.dot(t, wc2_ref[...], preferred_element_type=jnp.float32)
    r_ref[...] = z2
    wq_ref[...] = (w * inv).reshape(BE)


def _tc_edge(g, radp, edge_attr, w1e, wr, we2, be2, wc1, bc1, wc2):
    nblk = E // BE
    return pl.pallas_call(
        _tc_edge_body,
        grid=(nblk,),
        in_specs=[
            pl.BlockSpec((BE, D), lambda i: (i, 0)),
            pl.BlockSpec((BE,), lambda i: (i,)),
            pl.BlockSpec((DE, BE), lambda i: (0, i)),
            pl.BlockSpec((DE, D), lambda i: (0, 0)),
            pl.BlockSpec((1, D), lambda i: (0, 0)),
            pl.BlockSpec((D, D), lambda i: (0, 0)),
            pl.BlockSpec((1, D), lambda i: (0, 0)),
            pl.BlockSpec((D, D), lambda i: (0, 0)),
            pl.BlockSpec((1, D), lambda i: (0, 0)),
            pl.BlockSpec((D, 1), lambda i: (0, 0)),
        ],
        out_specs=[pl.BlockSpec((BE, D), lambda i: (i, 0)),
                   pl.BlockSpec((BE,), lambda i: (i,))],
        out_shape=[jax.ShapeDtypeStruct((E, D), jnp.float32),
                   jax.ShapeDtypeStruct((E,), jnp.float32)],
    )(g, radp, edge_attr, w1e, wr, we2, be2, wc1, bc1, wc2)


def _sc_scatter_body(r_hbm, wq_hbm, d3_hbm, dst_hbm, zh_hbm, zx_hbm,
                     outh_hbm, outx_hbm,
                     idx_v, bufh_v, bufx_v, wq_v, d3_v, acch_sh, accx_sh,
                     semin):
    c = lax.axis_index("c")
    s = lax.axis_index("s")
    base = (c * NS + s) * PER_TILE
    rsl = pl.ds(s * ROWS_PER_TILE, ROWS_PER_TILE)

    pltpu.sync_copy(zh_hbm.at[rsl], acch_sh.at[rsl])
    pltpu.sync_copy(zx_hbm.at[rsl], accx_sh.at[rsl])
    plsc.subcore_barrier()

    def zx(r, carry):
        bufx_v[r, pl.ds(0, 16)] = jnp.zeros((16,), jnp.float32)
        return carry
    lax.fori_loop(0, C, zx, 0)

    def issue(i, par):
        off = base + i * C
        pltpu.async_copy(dst_hbm.at[pl.ds(off, C)], idx_v.at[par],
                         semin.at[par])
        pltpu.async_copy(r_hbm.at[pl.ds(off, C)], bufh_v.at[par],
                         semin.at[par])
        pltpu.async_copy(wq_hbm.at[pl.ds(off, C)], wq_v.at[par],
                         semin.at[par])
        pltpu.async_copy(d3_hbm.at[pl.ds(off * 4, C * 4)], d3_v.at[par],
                         semin.at[par])

    def process(i, par):
        off = base + i * C
        pltpu.make_async_copy(dst_hbm.at[pl.ds(off, C)], idx_v.at[par],
                              semin.at[par]).wait()
        pltpu.make_async_copy(r_hbm.at[pl.ds(off, C)], bufh_v.at[par],
                              semin.at[par]).wait()
        pltpu.make_async_copy(wq_hbm.at[pl.ds(off, C)], wq_v.at[par],
                              semin.at[par]).wait()
        pltpu.make_async_copy(d3_hbm.at[pl.ds(off * 4, C * 4)],
                              d3_v.at[par], semin.at[par]).wait()

        def grp(g, c2):
            gsl = pl.ds(g * 16, 16)
            w16 = wq_v[par, gsl]
            l16 = lax.iota(jnp.int32, 16) + g * 16
            l4 = l16 * 4
            for k in range(3):
                dk = plsc.load_gather(d3_v.at[par], [l4 + k])
                plsc.store_scatter(bufx_v, [l16, jnp.full((16,), k, jnp.int32)],
                                   w16 * dk)
            return c2
        lax.fori_loop(0, C // 16, grp, 0)

        pltpu.sync_copy(bufh_v.at[par], acch_sh.at[idx_v.at[par]], add=True)
        pltpu.sync_copy(bufx_v, accx_sh.at[idx_v.at[par]], add=True)

    issue(0, 0)

    def pair(j, carry):
        i0 = j * 2
        issue(i0 + 1, 1)
        process(i0, 0)
        issue(i0 + 2, 0)
        process(i0 + 1, 1)
        return carry

    lax.fori_loop(0, NCHUNK // 2 - 1, pair, 0)
    i0 = NCHUNK - 3
    issue(i0 + 1, 1)
    process(i0, 0)
    issue(i0 + 2, 0)
    process(i0 + 1, 1)
    process(i0 + 2, 0)
    plsc.subcore_barrier()
    pltpu.sync_copy(acch_sh.at[rsl], outh_hbm.at[c, rsl])
    pltpu.sync_copy(accx_sh.at[rsl], outx_hbm.at[c, rsl])


def _sc_scatter(r, wq, d3, dst, zh, zx):
    mesh = plsc.VectorSubcoreMesh(core_axis_name="c", subcore_axis_name="s")
    fn = pl.kernel(
        _sc_scatter_body,
        out_type=[jax.ShapeDtypeStruct((NC, N, D), jnp.float32),
                  jax.ShapeDtypeStruct((NC, N, W2), jnp.float32)],
        mesh=mesh,
        compiler_params=_SC_PARAMS,
        scratch_types=[
            pltpu.VMEM((2, C), jnp.int32),
            pltpu.VMEM((2, C, D), jnp.float32),
            pltpu.VMEM((C, W2), jnp.float32),
            pltpu.VMEM((2, C), jnp.float32),
            pltpu.VMEM((2, 4 * C), jnp.float32),
            pltpu.VMEM_SHARED((N, D), jnp.float32),
            pltpu.VMEM_SHARED((N, W2), jnp.float32),
            pltpu.SemaphoreType.DMA((2,)),
        ],
    )
    return fn(r, wq, d3, dst, zh, zx)


def _tc_node_body(x_ref, coord_ref, acch_ref, accx_ref, wn1a_ref, wn1b_ref,
                  bn1_ref, wn2_ref, bn2_ref, wg_ref, bg_ref,
                  hout_ref, xout_ref, pooled_ref):
    hn = acch_ref[0] + acch_ref[1]
    xn = accx_ref[0][:, :3] + accx_ref[1][:, :3]
    h1 = _silu(jnp.dot(x_ref[...], wn1a_ref[...],
                       preferred_element_type=jnp.float32)
               + jnp.dot(hn, wn1b_ref[...],
                         preferred_element_type=jnp.float32)
               + bn1_ref[...])
    h_out = jnp.dot(h1, wn2_ref[...],
                    preferred_element_type=jnp.float32) + bn2_ref[...]
    g = jnp.dot(h_out, wg_ref[...],
                preferred_element_type=jnp.float32) + bg_ref[...]
    m = jnp.max(g)
    ex = jnp.exp(g - m)
    gate = ex / jnp.sum(ex)
    pooled_ref[...] = jnp.sum(gate * h_out, axis=0, keepdims=True)
    hout_ref[...] = h_out
    xout_ref[...] = coord_ref[...] + xn


def _tc_node(x, coord, acch, accx, wn1a, wn1b, bn1, wn2, bn2, wg, bg):
    return pl.pallas_call(
        _tc_node_body,
        out_shape=[jax.ShapeDtypeStruct((N, D), jnp.float32),
                   jax.ShapeDtypeStruct((N, 3), jnp.float32),
                   jax.ShapeDtypeStruct((1, D), jnp.float32)],
    )(x, coord, acch, accx, wn1a, wn1b, bn1, wn2, bn2, wg, bg)


def kernel(x, coord, edge_index, edge_attr, W_e1, b_e1, W_e2, b_e2,
           W_c1, b_c1, W_c2, W_n1, b_n1, W_n2, b_n2, W_g, b_g):
    w1a = W_e1[:D]
    w1b = W_e1[D:2 * D]
    wr = W_e1[2 * D:2 * D + 1]
    w1e = W_e1[2 * D + 1:]
    src = edge_index[0]
    dst = edge_index[1]
    coordp = jnp.pad(coord, ((0, 0), (0, 1))).reshape(-1)

    t1, t2 = _tc_prep(x, w1a, w1b, b_e1.reshape(1, D))
    g, d3, rad = _sc_gather(t1, t2, src, dst, coordp)
    r, wq = _tc_edge(g, rad, edge_attr.T, w1e, wr, W_e2,
                     b_e2.reshape(1, D), W_c1, b_c1.reshape(1, D), W_c2)
    zh = jnp.zeros((N, D), jnp.float32)
    zx = jnp.zeros((N, W2), jnp.float32)
    acch, accx = _sc_scatter(r, wq, d3, dst, zh, zx)
    h_out, x_out, pooled = _tc_node(
        x, coord, acch, accx, W_n1[:D], W_n1[D:], b_n1.reshape(1, D),
        W_n2, b_n2.reshape(1, D), W_g, b_g.reshape(1, 1))
    return (h_out, x_out, pooled)


# speedup vs baseline: 4.9960x; 1.0118x over previous
import jax
import jax.numpy as jnp
from jax import lax
from jax.experimental import pallas as pl
from jax.experimental.pallas import tpu as pltpu
from jax.experimental.pallas import tpu_sc as plsc

N = 10000
E = 320000
D = 128
DE = 16
NC, NS = 2, 16
NW = NC * NS
PER_TILE = E // NW
C = 80
NCHUNK = PER_TILE // C
BE = 512
W2 = 16
ROWS_PER_TILE = N // NS

_SC_PARAMS = pltpu.CompilerParams(use_tc_tiling_on_sc=False,
                                  needs_layout_passes=False)


def _silu(v):
    return v * jax.nn.sigmoid(v)


def _tc_prep_body(x_ref, w1a_ref, w1b_ref, b1_ref, t1_ref, t2_ref):
    xb = x_ref[...]
    t1_ref[...] = jnp.dot(xb, w1a_ref[...],
                          preferred_element_type=jnp.float32) + b1_ref[...]
    t2_ref[...] = jnp.dot(xb, w1b_ref[...],
                          preferred_element_type=jnp.float32)


def _tc_prep(x, w1a, w1b, b1):
    return pl.pallas_call(
        _tc_prep_body,
        out_shape=[jax.ShapeDtypeStruct((N, D), jnp.float32),
                   jax.ShapeDtypeStruct((N, D), jnp.float32)],
    )(x, w1a, w1b, b1)


def _sc_gather_body(t1_hbm, t2_hbm, src_hbm, dst_hbm, coord_hbm,
                    g_hbm, d3_hbm, rad_hbm,
                    idx1_v, idx2_v, buf1_v, buf2_v, coord_v, mp_v, rad_v,
                    sem1, sem2, semwb):
    wid = lax.axis_index("s") * NC + lax.axis_index("c")
    base = wid * PER_TILE

    pltpu.sync_copy(coord_hbm, coord_v)

    def issue(i, par):
        off = base + i * C
        pltpu.sync_copy(src_hbm.at[pl.ds(off, C)], idx1_v.at[par])
        pltpu.sync_copy(dst_hbm.at[pl.ds(off, C)], idx2_v.at[par])
        pltpu.async_copy(t1_hbm.at[idx1_v.at[par]], buf1_v.at[par],
                         sem1.at[par])
        pltpu.async_copy(t2_hbm.at[idx2_v.at[par]], buf2_v.at[par],
                         sem2.at[par])

    def drain_wb(i, par):
        off = base + i * C
        pltpu.make_async_copy(mp_v.at[par], d3_hbm.at[pl.ds(off * 4, C * 4)],
                              semwb.at[par]).wait()
        pltpu.make_async_copy(rad_v.at[par], rad_hbm.at[pl.ds(off, C)],
                              semwb.at[par]).wait()

    def process(i, par, drain):
        off = base + i * C
        pltpu.make_async_copy(t1_hbm.at[idx1_v.at[par]],
                              buf1_v.at[par], sem1.at[par]).wait()
        pltpu.make_async_copy(t2_hbm.at[idx2_v.at[par]],
                              buf2_v.at[par], sem2.at[par]).wait()
        if drain:
            drain_wb(i - 2, par)

        def row(r, c2):
            for cc in range(D // 16):
                sl = pl.ds(cc * 16, 16)
                buf1_v[par, r, sl] = buf1_v[par, r, sl] + buf2_v[par, r, sl]
            return c2
        lax.fori_loop(0, C, row, 0)

        def grp(g, c3):
            gsl = pl.ds(g * 16, 16)
            s16 = idx1_v[par, gsl] * 4
            d16 = idx2_v[par, gsl] * 4
            l16 = (lax.iota(jnp.int32, 16) + g * 16) * 4
            dif = []
            for k in range(3):
                cs = plsc.load_gather(coord_v, [s16 + k])
                cd = plsc.load_gather(coord_v, [d16 + k])
                dif.append(cs - cd)
                plsc.store_scatter(mp_v.at[par], [l16 + k], dif[k])
            rad_v[par, gsl] = (dif[0] * dif[0] + dif[1] * dif[1]
                               + dif[2] * dif[2])
            return c3
        lax.fori_loop(0, C // 16, grp, 0)

        pltpu.sync_copy(buf1_v.at[par], g_hbm.at[pl.ds(off, C)])
        pltpu.async_copy(mp_v.at[par], d3_hbm.at[pl.ds(off * 4, C * 4)],
                         semwb.at[par])
        pltpu.async_copy(rad_v.at[par], rad_hbm.at[pl.ds(off, C)],
                         semwb.at[par])

    issue(0, 0)
    issue(1, 1)
    process(0, 0, False)
    issue(2, 0)
    process(1, 1, False)

    def pair(j, carry):
        i0 = j * 2
        issue(i0 + 1, 1)
        process(i0, 0, True)
        issue(i0 + 2, 0)
        process(i0 + 1, 1, True)
        return carry

    lax.fori_loop(1, NCHUNK // 2 - 1, pair, 0)
    i0 = NCHUNK - 3
    issue(i0 + 1, 1)
    process(i0, 0, True)
    issue(i0 + 2, 0)
    process(i0 + 1, 1, True)
    process(i0 + 2, 0, True)
    drain_wb(NCHUNK - 2, 1)
    drain_wb(NCHUNK - 1, 0)


def _sc_gather(t1, t2, src, dst, coordp):
    mesh = plsc.VectorSubcoreMesh(core_axis_name="c", subcore_axis_name="s")
    fn = pl.kernel(
        _sc_gather_body,
        out_type=[jax.ShapeDtypeStruct((E, D), jnp.float32),
                  jax.ShapeDtypeStruct((4 * E,), jnp.float32),
                  jax.ShapeDtypeStruct((E,), jnp.float32)],
        mesh=mesh,
        compiler_params=_SC_PARAMS,
        scratch_types=[
            pltpu.VMEM((2, C), jnp.int32),
            pltpu.VMEM((2, C), jnp.int32),
            pltpu.VMEM((2, C, D), jnp.float32),
            pltpu.VMEM((2, C, D), jnp.float32),
            pltpu.VMEM((4 * N,), jnp.float32),
            pltpu.VMEM((2, 4 * C), jnp.float32),
            pltpu.VMEM((2, C), jnp.float32),
            pltpu.SemaphoreType.DMA((2,)),
            pltpu.SemaphoreType.DMA((2,)),
            pltpu.SemaphoreType.DMA((2,)),
        ],
    )
    return fn(t1, t2, src, dst, coordp)


def _tc_edge_body(g_ref, radp_ref, ea_ref, w1e_ref, wr_ref, we2_ref, be2_ref,
                  wc1_ref, bc1_ref, wc2_ref, r_ref, wq_ref):
    radial = radp_ref[...].reshape(BE, 1)
    inv = 1.0 / (jnp.sqrt(radial) + 1e-30)
    eac = lax.dot_general(ea_ref[...], w1e_ref[...],
                          (((0,), (0,)), ((), ())),
                          preferred_element_type=jnp.float32)
    z1 = g_ref[...] + radial * wr_ref[...] + eac
    z1 = _silu(z1)
    z2 = _silu(jnp.dot(z1, we2_ref[...],
                       preferred_element_type=jnp.float32) + be2_ref[...])
    t = _silu(jnp.dot(z2, wc1_ref[...],
                      preferred_element_type=jnp.float32) + bc1_ref[...])
    w = jnp.dot(t, wc2_ref[...], preferred_element_type=jnp.float32)
    r_ref[...] = z2
    wq_ref[...] = (w * inv).reshape(BE)


def _tc_edge(g, radp, edge_attr, w1e, wr, we2, be2, wc1, bc1, wc2):
    nblk = E // BE
    return pl.pallas_call(
        _tc_edge_body,
        grid=(nblk,),
        in_specs=[
            pl.BlockSpec((BE, D), lambda i: (i, 0)),
            pl.BlockSpec((BE,), lambda i: (i,)),
            pl.BlockSpec((DE, BE), lambda i: (0, i)),
            pl.BlockSpec((DE, D), lambda i: (0, 0)),
            pl.BlockSpec((1, D), lambda i: (0, 0)),
            pl.BlockSpec((D, D), lambda i: (0, 0)),
            pl.BlockSpec((1, D), lambda i: (0, 0)),
            pl.BlockSpec((D, D), lambda i: (0, 0)),
            pl.BlockSpec((1, D), lambda i: (0, 0)),
            pl.BlockSpec((D, 1), lambda i: (0, 0)),
        ],
        out_specs=[pl.BlockSpec((BE, D), lambda i: (i, 0)),
                   pl.BlockSpec((BE,), lambda i: (i,))],
        out_shape=[jax.ShapeDtypeStruct((E, D), jnp.float32),
                   jax.ShapeDtypeStruct((E,), jnp.float32)],
    )(g, radp, edge_attr, w1e, wr, we2, be2, wc1, bc1, wc2)


def _sc_scatter_body(r_hbm, wq_hbm, d3_hbm, dst_hbm, zh_hbm, zx_hbm,
                     outh_hbm, outx_hbm,
                     idx_v, bufh_v, bufx_v, wq_v, d3_v, acch_sh, accx_sh,
                     semin):
    c = lax.axis_index("c")
    s = lax.axis_index("s")
    base = (c * NS + s) * PER_TILE
    rsl = pl.ds(s * ROWS_PER_TILE, ROWS_PER_TILE)

    pltpu.sync_copy(zh_hbm.at[rsl], acch_sh.at[rsl])
    pltpu.sync_copy(zx_hbm.at[rsl], accx_sh.at[rsl])
    plsc.subcore_barrier()

    def zx(r, carry):
        bufx_v[r, pl.ds(0, 16)] = jnp.zeros((16,), jnp.float32)
        return carry
    lax.fori_loop(0, C, zx, 0)

    def issue(i, par):
        off = base + i * C
        pltpu.async_copy(dst_hbm.at[pl.ds(off, C)], idx_v.at[par],
                         semin.at[par])
        pltpu.async_copy(r_hbm.at[pl.ds(off, C)], bufh_v.at[par],
                         semin.at[par])
        pltpu.async_copy(wq_hbm.at[pl.ds(off, C)], wq_v.at[par],
                         semin.at[par])
        pltpu.async_copy(d3_hbm.at[pl.ds(off * 4, C * 4)], d3_v.at[par],
                         semin.at[par])

    def process(i, par):
        off = base + i * C
        pltpu.make_async_copy(dst_hbm.at[pl.ds(off, C)], idx_v.at[par],
                              semin.at[par]).wait()
        pltpu.make_async_copy(r_hbm.at[pl.ds(off, C)], bufh_v.at[par],
                              semin.at[par]).wait()
        pltpu.make_async_copy(wq_hbm.at[pl.ds(off, C)], wq_v.at[par],
                              semin.at[par]).wait()
        pltpu.make_async_copy(d3_hbm.at[pl.ds(off * 4, C * 4)],
                              d3_v.at[par], semin.at[par]).wait()

        def grp(g, c2):
            gsl = pl.ds(g * 16, 16)
            w16 = wq_v[par, gsl]
            l16 = lax.iota(jnp.int32, 16) + g * 16
            l4 = l16 * 4
            for k in range(3):
                dk = plsc.load_gather(d3_v.at[par], [l4 + k])
                plsc.store_scatter(bufx_v, [l16, jnp.full((16,), k, jnp.int32)],
                                   w16 * dk)
            return c2
        lax.fori_loop(0, C // 16, grp, 0)

        pltpu.sync_copy(bufh_v.at[par], acch_sh.at[idx_v.at[par]], add=True)
        pltpu.sync_copy(bufx_v, accx_sh.at[idx_v.at[par]], add=True)

    issue(0, 0)

    def pair(j, carry):
        i0 = j * 2
        issue(i0 + 1, 1)
        process(i0, 0)
        issue(i0 + 2, 0)
        process(i0 + 1, 1)
        return carry

    lax.fori_loop(0, NCHUNK // 2 - 1, pair, 0)
    i0 = NCHUNK - 3
    issue(i0 + 1, 1)
    process(i0, 0)
    issue(i0 + 2, 0)
    process(i0 + 1, 1)
    process(i0 + 2, 0)
    plsc.subcore_barrier()
    pltpu.sync_copy(acch_sh.at[rsl], outh_hbm.at[c, rsl])
    pltpu.sync_copy(accx_sh.at[rsl], outx_hbm.at[c, rsl])


def _sc_scatter(r, wq, d3, dst, zh, zx):
    mesh = plsc.VectorSubcoreMesh(core_axis_name="c", subcore_axis_name="s")
    fn = pl.kernel(
        _sc_scatter_body,
        out_type=[jax.ShapeDtypeStruct((NC, N, D), jnp.float32),
                  jax.ShapeDtypeStruct((NC, N, W2), jnp.float32)],
        mesh=mesh,
        compiler_params=_SC_PARAMS,
        scratch_types=[
            pltpu.VMEM((2, C), jnp.int32),
            pltpu.VMEM((2, C, D), jnp.float32),
            pltpu.VMEM((C, W2), jnp.float32),
            pltpu.VMEM((2, C), jnp.float32),
            pltpu.VMEM((2, 4 * C), jnp.float32),
            pltpu.VMEM_SHARED((N, D), jnp.float32),
            pltpu.VMEM_SHARED((N, W2), jnp.float32),
            pltpu.SemaphoreType.DMA((2,)),
        ],
    )
    return fn(r, wq, d3, dst, zh, zx)


def _tc_node_body(x_ref, coord_ref, acch_ref, accx_ref, wn1a_ref, wn1b_ref,
                  bn1_ref, wn2_ref, bn2_ref, wg_ref, bg_ref,
                  hout_ref, xout_ref, pooled_ref):
    hn = acch_ref[0] + acch_ref[1]
    xn = accx_ref[0][:, :3] + accx_ref[1][:, :3]
    h1 = _silu(jnp.dot(x_ref[...], wn1a_ref[...],
                       preferred_element_type=jnp.float32)
               + jnp.dot(hn, wn1b_ref[...],
                         preferred_element_type=jnp.float32)
               + bn1_ref[...])
    h_out = jnp.dot(h1, wn2_ref[...],
                    preferred_element_type=jnp.float32) + bn2_ref[...]
    g = jnp.dot(h_out, wg_ref[...],
                preferred_element_type=jnp.float32) + bg_ref[...]
    m = jnp.max(g)
    ex = jnp.exp(g - m)
    gate = ex / jnp.sum(ex)
    pooled_ref[...] = jnp.sum(gate * h_out, axis=0, keepdims=True)
    hout_ref[...] = h_out
    xout_ref[...] = coord_ref[...] + xn


def _tc_node(x, coord, acch, accx, wn1a, wn1b, bn1, wn2, bn2, wg, bg):
    return pl.pallas_call(
        _tc_node_body,
        out_shape=[jax.ShapeDtypeStruct((N, D), jnp.float32),
                   jax.ShapeDtypeStruct((N, 3), jnp.float32),
                   jax.ShapeDtypeStruct((1, D), jnp.float32)],
    )(x, coord, acch, accx, wn1a, wn1b, bn1, wn2, bn2, wg, bg)


def kernel(x, coord, edge_index, edge_attr, W_e1, b_e1, W_e2, b_e2,
           W_c1, b_c1, W_c2, W_n1, b_n1, W_n2, b_n2, W_g, b_g):
    w1a = W_e1[:D]
    w1b = W_e1[D:2 * D]
    wr = W_e1[2 * D:2 * D + 1]
    w1e = W_e1[2 * D + 1:]
    src = edge_index[0]
    dst = edge_index[1]
    coordp = jnp.pad(coord, ((0, 0), (0, 1))).reshape(-1)

    t1, t2 = _tc_prep(x, w1a, w1b, b_e1.reshape(1, D))
    g, d3, rad = _sc_gather(t1, t2, src, dst, coordp)
    r, wq = _tc_edge(g, rad, edge_attr.T, w1e, wr, W_e2,
                     b_e2.reshape(1, D), W_c1, b_c1.reshape(1, D), W_c2)
    zh = jnp.zeros((N, D), jnp.float32)
    zx = jnp.zeros((N, W2), jnp.float32)
    acch, accx = _sc_scatter(r, wq, d3, dst, zh, zx)
    h_out, x_out, pooled = _tc_node(
        x, coord, acch, accx, W_n1[:D], W_n1[D:], b_n1.reshape(1, D),
        W_n2, b_n2.reshape(1, D), W_g, b_g.reshape(1, 1))
    return (h_out, x_out, pooled)
